# Initial kernel scaffold; baseline (speedup 1.0000x reference)
import jax
import jax.numpy as jnp
from jax.experimental import pallas as pl


def kernel(preds_S, preds_T, gt_labels):
    raise NotImplementedError("write your pallas kernel here")


# speedup vs baseline: 6.9217x; 6.9217x over previous
import functools

import jax
import jax.numpy as jnp
from jax.experimental import pallas as pl
from jax.experimental.pallas import tpu as pltpu

_TAU = 1.0
_C = 14
_LOSS_WEIGHT = 1.0


def _kl_loss_kernel(gt_ref, s_ref, t_ref, out_ref, acc_s, acc_c, *, n_pblk, n_b):
    b = pl.program_id(0)
    ip = pl.program_id(1)

    @pl.when(jnp.logical_and(b == 0, ip == 0))
    def _init():
        acc_s[...] = jnp.zeros_like(acc_s)
        acc_c[...] = jnp.zeros_like(acc_c)

    s = s_ref[0]
    t = t_ref[0]

    s_max = jnp.max(s, axis=0, keepdims=True)
    s_sh = s - s_max
    s_lse = jnp.log(jnp.sum(jnp.exp(s_sh), axis=0, keepdims=True))
    log_s = s_sh - s_lse

    t_max = jnp.max(t, axis=0, keepdims=True)
    t_sh = t - t_max
    t_exp = jnp.exp(t_sh)
    t_sum = jnp.sum(t_exp, axis=0, keepdims=True)
    t_p = t_exp / t_sum
    log_t = t_sh - jnp.log(t_sum)

    kl_pix = jnp.sum(t_p * (log_t - log_s), axis=0, keepdims=True)

    gt = gt_ref[0]
    cls = jax.lax.broadcasted_iota(jnp.int32, (_C, kl_pix.shape[1]), 0)
    onehot = (gt == cls).astype(jnp.float32)

    sums_part = jnp.sum(kl_pix * onehot, axis=1, keepdims=True)
    cnts_part = jnp.sum(onehot, axis=1, keepdims=True)

    row = pl.ds(b * 16, _C)
    acc_s[row, 0:1] += sums_part
    acc_c[row, 0:1] += cnts_part

    @pl.when(jnp.logical_and(b == n_b - 1, ip == n_pblk - 1))
    def _finish():
        sums = acc_s[:, 0:1]
        cnts = acc_c[:, 0:1]
        rid = jax.lax.broadcasted_iota(jnp.int32, sums.shape, 0)
        cid = jax.lax.bitwise_and(rid, 15)
        valid = jnp.logical_and(cid >= 1, cid <= _C - 1)
        valid = jnp.logical_and(valid, cnts > 0.0)
        per = jnp.where(valid, sums / (_C * jnp.maximum(cnts, 1.0)), 0.0)
        out_ref[...] = jnp.sum(per, axis=0, keepdims=True) * (_TAU ** 2) * _LOSS_WEIGHT


def kernel(preds_S, preds_T, gt_labels):
    B, C, H, W = preds_S.shape
    P = H * W
    BLK_P = 32768
    n_pblk = P // BLK_P

    s3 = preds_S.reshape(B, C, P)
    t3 = preds_T.reshape(B, C, P)
    gt3 = gt_labels.reshape(B, 1, P).astype(jnp.int32)

    out = pl.pallas_call(
        functools.partial(_kl_loss_kernel, n_pblk=n_pblk, n_b=B),
        grid=(B, n_pblk),
        in_specs=[
            pl.BlockSpec((1, 1, BLK_P), lambda b, ip: (b, 0, ip)),
            pl.BlockSpec((1, C, BLK_P), lambda b, ip: (b, 0, ip)),
            pl.BlockSpec((1, C, BLK_P), lambda b, ip: (b, 0, ip)),
        ],
        out_specs=pl.BlockSpec((1, 1), lambda b, ip: (0, 0)),
        out_shape=jax.ShapeDtypeStruct((1, 1), jnp.float32),
        scratch_shapes=[
            pltpu.VMEM((64, 128), jnp.float32),
            pltpu.VMEM((64, 128), jnp.float32),
        ],
        compiler_params=pltpu.CompilerParams(
            dimension_semantics=("arbitrary", "arbitrary"),
        ),
    )(gt3, s3, t3)
    return out[0, 0]


# speedup vs baseline: 32.3776x; 4.6777x over previous
import functools

import jax
import jax.numpy as jnp
from jax.experimental import pallas as pl
from jax.experimental.pallas import tpu as pltpu

_TAU = 1.0
_C = 14
_LOSS_WEIGHT = 1.0


def _fold(x):
    h, w = x.shape
    while w > 128:
        w //= 2
        x = x[:, :w] + x[:, w:]
    while h > 8:
        h //= 2
        x = x[:h, :] + x[h:, :]
    return x


def _kl_loss_kernel(gt_ref, s_ref, t_ref, out_ref, acc_s, acc_c, *, n_hblk, n_b):
    b = pl.program_id(0)
    ih = pl.program_id(1)

    @pl.when(jnp.logical_and(b == 0, ih == 0))
    def _init():
        acc_s[...] = jnp.zeros_like(acc_s)
        acc_c[...] = jnp.zeros_like(acc_c)

    s = s_ref[0]
    t = t_ref[0]
    if _TAU != 1.0:
        s = s / _TAU
        t = t / _TAU

    es = jnp.exp(s)
    et = jnp.exp(t)
    w = et * (t - s)

    se = jnp.sum(es, axis=0)
    te = jnp.sum(et, axis=0)
    we = jnp.sum(w, axis=0)

    kl = we / te + jnp.log(se) - jnp.log(te)

    gt = gt_ref[0, 0]
    ones = jnp.ones_like(kl)
    for c in range(_C):
        m = gt == c
        fk = _fold(jnp.where(m, kl, 0.0))
        fc = _fold(jnp.where(m, ones, 0.0))
        row = pl.ds(b * 128 + c * 8, 8)
        acc_s[row, :] += fk
        acc_c[row, :] += fc

    @pl.when(jnp.logical_and(b == n_b - 1, ih == n_hblk - 1))
    def _finish():
        sums3 = acc_s[...].reshape(64, 8, 128)
        cnts3 = acc_c[...].reshape(64, 8, 128)
        sums = jnp.sum(jnp.sum(sums3, axis=1), axis=1, keepdims=True)
        cnts = jnp.sum(jnp.sum(cnts3, axis=1), axis=1, keepdims=True)
        rid = jax.lax.broadcasted_iota(jnp.int32, sums.shape, 0)
        cid = jax.lax.bitwise_and(rid, 15)
        valid = jnp.logical_and(cid >= 1, cid <= _C - 1)
        valid = jnp.logical_and(valid, cnts > 0.0)
        per = jnp.where(valid, sums / (_C * jnp.maximum(cnts, 1.0)), 0.0)
        out_ref[...] = jnp.sum(per, axis=0, keepdims=True) * (_TAU ** 2) * _LOSS_WEIGHT


def kernel(preds_S, preds_T, gt_labels):
    B, C, H, W = preds_S.shape
    BLK_H = 64
    n_hblk = H // BLK_H

    gt = gt_labels.astype(jnp.int32)

    out = pl.pallas_call(
        functools.partial(_kl_loss_kernel, n_hblk=n_hblk, n_b=B),
        grid=(B, n_hblk),
        in_specs=[
            pl.BlockSpec((1, 1, BLK_H, W), lambda b, ih: (b, 0, ih, 0)),
            pl.BlockSpec((1, C, BLK_H, W), lambda b, ih: (b, 0, ih, 0)),
            pl.BlockSpec((1, C, BLK_H, W), lambda b, ih: (b, 0, ih, 0)),
        ],
        out_specs=pl.BlockSpec((1, 1), lambda b, ih: (0, 0)),
        out_shape=jax.ShapeDtypeStruct((1, 1), jnp.float32),
        scratch_shapes=[
            pltpu.VMEM((512, 128), jnp.float32),
            pltpu.VMEM((512, 128), jnp.float32),
        ],
        compiler_params=pltpu.CompilerParams(
            dimension_semantics=("arbitrary", "arbitrary"),
        ),
    )(gt, preds_S, preds_T)
    return out[0, 0]


# speedup vs baseline: 35.3765x; 1.0926x over previous
import functools

import jax
import jax.numpy as jnp
from jax.experimental import pallas as pl
from jax.experimental.pallas import tpu as pltpu

_TAU = 1.0
_C = 14
_LOSS_WEIGHT = 1.0


def _fold(x):
    h, w = x.shape
    while w > 128:
        w //= 2
        x = x[:, :w] + x[:, w:]
    while h > 8:
        h //= 2
        x = x[:h, :] + x[h:, :]
    return x


def _kl_loss_kernel(gt_ref, s_ref, t_ref, out_ref, acc_s, acc_c, *, n_hblk, n_b):
    b = pl.program_id(0)
    ih = pl.program_id(1)

    @pl.when(jnp.logical_and(b == 0, ih == 0))
    def _init():
        acc_s[...] = jnp.zeros_like(acc_s)
        acc_c[...] = jnp.zeros_like(acc_c)

    se = None
    for c in range(_C):
        sc = s_ref[0, c]
        tc = t_ref[0, c]
        if _TAU != 1.0:
            sc = sc / _TAU
            tc = tc / _TAU
        esc = jnp.exp(sc)
        etc = jnp.exp(tc)
        wc = etc * (tc - sc)
        if se is None:
            se, te, we = esc, etc, wc
        else:
            se = se + esc
            te = te + etc
            we = we + wc

    kl = we / te + jnp.log(se) - jnp.log(te)

    gt = gt_ref[0, 0]
    ones = jnp.ones_like(kl)
    for c in range(1, _C):
        m = gt == c
        fk = _fold(jnp.where(m, kl, 0.0))
        fc = _fold(jnp.where(m, ones, 0.0))
        row = pl.ds(b * 128 + c * 8, 8)
        acc_s[row, :] += fk
        acc_c[row, :] += fc

    @pl.when(jnp.logical_and(b == n_b - 1, ih == n_hblk - 1))
    def _finish():
        sums3 = acc_s[...].reshape(64, 8, 128)
        cnts3 = acc_c[...].reshape(64, 8, 128)
        sums = jnp.sum(jnp.sum(sums3, axis=1), axis=1, keepdims=True)
        cnts = jnp.sum(jnp.sum(cnts3, axis=1), axis=1, keepdims=True)
        rid = jax.lax.broadcasted_iota(jnp.int32, sums.shape, 0)
        cid = jax.lax.bitwise_and(rid, 15)
        valid = jnp.logical_and(cid >= 1, cid <= _C - 1)
        valid = jnp.logical_and(valid, cnts > 0.0)
        per = jnp.where(valid, sums / (_C * jnp.maximum(cnts, 1.0)), 0.0)
        out_ref[...] = jnp.sum(per, axis=0, keepdims=True) * (_TAU ** 2) * _LOSS_WEIGHT


def kernel(preds_S, preds_T, gt_labels):
    B, C, H, W = preds_S.shape
    BLK_H = 64
    n_hblk = H // BLK_H

    gt = gt_labels.astype(jnp.int32)

    out = pl.pallas_call(
        functools.partial(_kl_loss_kernel, n_hblk=n_hblk, n_b=B),
        grid=(B, n_hblk),
        in_specs=[
            pl.BlockSpec((1, 1, BLK_H, W), lambda b, ih: (b, 0, ih, 0)),
            pl.BlockSpec((1, C, BLK_H, W), lambda b, ih: (b, 0, ih, 0)),
            pl.BlockSpec((1, C, BLK_H, W), lambda b, ih: (b, 0, ih, 0)),
        ],
        out_specs=pl.BlockSpec((1, 1), lambda b, ih: (0, 0)),
        out_shape=jax.ShapeDtypeStruct((1, 1), jnp.float32),
        scratch_shapes=[
            pltpu.VMEM((512, 128), jnp.float32),
            pltpu.VMEM((512, 128), jnp.float32),
        ],
        compiler_params=pltpu.CompilerParams(
            dimension_semantics=("arbitrary", "arbitrary"),
        ),
    )(gt, preds_S, preds_T)
    return out[0, 0]


# speedup vs baseline: 37.3588x; 1.0560x over previous
import functools

import jax
import jax.numpy as jnp
from jax.experimental import pallas as pl
from jax.experimental.pallas import tpu as pltpu

_TAU = 1.0
_C = 14
_LOSS_WEIGHT = 1.0


def _fold(x):
    h, w = x.shape
    while w > 128:
        w //= 2
        x = x[:, :w] + x[:, w:]
    while h > 8:
        h //= 2
        x = x[:h, :] + x[h:, :]
    return x


def _kl_loss_kernel(gt_ref, s_ref, t_ref, out_ref, acc_s, acc_c, *, n_hblk, n_b):
    b = pl.program_id(0)
    ih = pl.program_id(1)

    @pl.when(jnp.logical_and(b == 0, ih == 0))
    def _init():
        acc_s[...] = jnp.zeros_like(acc_s)
        acc_c[...] = jnp.zeros_like(acc_c)

    se = None
    for c in range(_C):
        sc = s_ref[0, c]
        tc = t_ref[0, c]
        if _TAU != 1.0:
            sc = sc / _TAU
            tc = tc / _TAU
        esc = jnp.exp(sc)
        etc = jnp.exp(tc)
        wc = etc * (tc - sc)
        if se is None:
            se, te, we = esc, etc, wc
        else:
            se = se + esc
            te = te + etc
            we = we + wc

    kl = we / te + jnp.log(se) - jnp.log(te)

    gt = gt_ref[0, 0]
    ones = jnp.ones_like(kl)
    for c in range(1, _C):
        m = gt == c
        fk = _fold(jnp.where(m, kl, 0.0))
        fc = _fold(jnp.where(m, ones, 0.0))
        row = pl.ds(b * 128 + c * 8, 8)
        acc_s[row, :] += fk
        acc_c[row, :] += fc

    @pl.when(jnp.logical_and(b == n_b - 1, ih == n_hblk - 1))
    def _finish():
        sums3 = acc_s[...].reshape(64, 8, 128)
        cnts3 = acc_c[...].reshape(64, 8, 128)
        sums = jnp.sum(jnp.sum(sums3, axis=1), axis=1, keepdims=True)
        cnts = jnp.sum(jnp.sum(cnts3, axis=1), axis=1, keepdims=True)
        rid = jax.lax.broadcasted_iota(jnp.int32, sums.shape, 0)
        cid = jax.lax.bitwise_and(rid, 15)
        valid = jnp.logical_and(cid >= 1, cid <= _C - 1)
        valid = jnp.logical_and(valid, cnts > 0.0)
        per = jnp.where(valid, sums / (_C * jnp.maximum(cnts, 1.0)), 0.0)
        out_ref[...] = jnp.sum(per, axis=0, keepdims=True) * (_TAU ** 2) * _LOSS_WEIGHT


def kernel(preds_S, preds_T, gt_labels):
    B, C, H, W = preds_S.shape
    BLK_H = 128
    n_hblk = H // BLK_H

    gt = gt_labels.astype(jnp.int32)

    out = pl.pallas_call(
        functools.partial(_kl_loss_kernel, n_hblk=n_hblk, n_b=B),
        grid=(B, n_hblk),
        in_specs=[
            pl.BlockSpec((1, 1, BLK_H, W), lambda b, ih: (b, 0, ih, 0)),
            pl.BlockSpec((1, C, BLK_H, W), lambda b, ih: (b, 0, ih, 0)),
            pl.BlockSpec((1, C, BLK_H, W), lambda b, ih: (b, 0, ih, 0)),
        ],
        out_specs=pl.BlockSpec((1, 1), lambda b, ih: (0, 0)),
        out_shape=jax.ShapeDtypeStruct((1, 1), jnp.float32),
        scratch_shapes=[
            pltpu.VMEM((512, 128), jnp.float32),
            pltpu.VMEM((512, 128), jnp.float32),
        ],
        compiler_params=pltpu.CompilerParams(
            dimension_semantics=("arbitrary", "arbitrary"),
        ),
    )(gt, preds_S, preds_T)
    return out[0, 0]
